# baseline (speedup 1.0000x reference)
import jax
import jax.numpy as jnp
from jax import lax
from jax.experimental import pallas as pl
from jax.experimental.pallas import tpu as pltpu
from jax.experimental.pallas import tpu_sc as plsc

B = 16384
H = 64
NC, NS = 2, 16
NW = NC * NS
TPW = (B // 128) // NW
R = 128
PITCH = 129

LN2 = 0.6931471805599453
SQRT2 = 1.4142135623730951
C_BING = 0.6931471805599453 + 1.5 * 1.1447298858494002
LOG2PI3 = 3.0 * 1.8378770664093453


def _ln(x):
    bits = plsc.bitcast(x, jnp.int32)
    e = (bits >> 23) - 127
    m = plsc.bitcast((bits & 0x007FFFFF) | 0x3F800000, jnp.float32)
    big = m > SQRT2
    m = jnp.where(big, m * 0.5, m)
    ef = e.astype(jnp.float32) + jnp.where(big, 1.0, 0.0)
    t = (m - 1.0) / (m + 1.0)
    t2 = t * t
    poly = t * (2.0 + t2 * (0.66666667 + t2 * (0.4 + t2 * (0.28571429 + t2 * 0.22222222))))
    return ef * LN2 + poly


def _rsqrt(x):
    bits = plsc.bitcast(x, jnp.int32)
    y = plsc.bitcast(0x5F3759DF - (bits >> 1), jnp.float32)
    for _ in range(3):
        y = y * (1.5 - 0.5 * x * y * y)
    return y


def _body(q_h, l_h, w_h, gq_h, x_h, v_h, gt_h, km_h, out_h, *scr):
    qv, lv, xv, vv, wv, gqv, gtv, kmv, outv, lbuf, selv, semA, semB = scr

    cid = lax.axis_index("c")
    sid = lax.axis_index("s")
    wid = sid * NC + cid

    pltpu.sync_copy(km_h, kmv)
    km = kmv[...]

    iota = lax.broadcasted_iota(jnp.int32, (16,), 0)
    iotap = iota * PITCH
    lane_lt8 = iota < 8
    zero16 = jnp.zeros((16,), jnp.int32)

    def splat(i):
        return zero16 + i

    def phase_a(m, carry):
        rv = m * 16 + iota
        gq = [plsc.load_gather(gqv, [splat(c), rv]) for c in range(4)]
        gt = [plsc.load_gather(gtv, [splat(c), rv]) for c in range(3)]
        for h in range(H):
            ht, hs = h >> 3, h & 7
            dq = [jnp.abs(plsc.load_gather(qv, [splat(h), splat(c), rv]) - gq[c])
                  for c in range(4)]
            dx = [jnp.abs(plsc.load_gather(xv, [splat(c), splat(ht), splat(hs), rv]) - gt[c])
                  for c in range(3)]
            acc = ((dq[0] + dq[1]) + (dq[2] + dq[3])) + ((dx[0] + dx[1]) + dx[2])
            plsc.store_scatter(lbuf, [rv + h * PITCH], acc)
        return carry

    def phase_w(m, accb):
        rv = m * 16 + iota
        uprod = jnp.zeros((16,), jnp.float32) + 1.0
        relu = jnp.zeros((16,), jnp.float32)
        for h in range(H):
            ht, hs = h >> 3, h & 7
            wj = plsc.load_gather(wv, [splat(ht), splat(hs), rv])
            relu = relu + 0.5 * (wj + jnp.abs(wj))
            uprod = uprod * (1.0 + jnp.exp(-jnp.abs(wj)))
            if (h & 15) == 15:
                accb = accb + _ln(uprod)
                uprod = jnp.zeros((16,), jnp.float32) + 1.0
        return accb + relu

    def phase_b(r, carry):
        tk = None
        tv = None
        for j in range(4):
            keys = plsc.load_gather(lbuf, [iotap + (j * 16 * PITCH + r)])
            sk, sv = plsc.sort_key_val(keys, iota + 16 * j)
            if tk is None:
                tk, tv = sk, sv
            else:
                ck = jnp.where(lane_lt8, tk, jnp.flip(sk))
                cv = jnp.where(lane_lt8, tv, jnp.flip(sv))
                tk, tv = plsc.sort_key_val(ck, cv)
        plsc.store_scatter(selv, [iotap + r], tv, mask=lane_lt8)
        return carry

    def phase_c(m, accs):
        accp, accg, accw = accs
        rv = m * 16 + iota
        gq = [plsc.load_gather(gqv, [splat(c), rv]) for c in range(4)]
        gt = [plsc.load_gather(gtv, [splat(c), rv]) for c in range(3)]
        pprod = jnp.zeros((16,), jnp.float32) + 1.0
        dprod = jnp.zeros((16,), jnp.float32) + 1.0
        for s in range(8):
            kms = km[s]
            hsel = plsc.load_gather(selv, [rv + s * PITCH])
            hst = hsel >> 3
            hss = hsel & 7

            qs = [plsc.load_gather(qv, [hsel, splat(c), rv]) for c in range(4)]
            sq = qs[0] * qs[0] + qs[1] * qs[1] + qs[2] * qs[2] + qs[3] * qs[3]
            rn = _rsqrt(sq)
            a_ = qs[0] * rn
            b_ = qs[1] * rn
            c_ = qs[2] * rn
            d_ = qs[3] * rn
            t1 = gq[1] * a_ - gq[0] * b_ + gq[3] * c_ - gq[2] * d_
            t2 = gq[2] * a_ - gq[3] * b_ - gq[0] * c_ + gq[1] * d_
            t3 = gq[3] * a_ + gq[2] * b_ - gq[1] * c_ - gq[0] * d_
            p0 = jnp.abs(plsc.load_gather(lv, [splat(0), hst, hss, rv])) + 1e-6
            p1 = jnp.abs(plsc.load_gather(lv, [splat(1), hst, hss, rv])) + 1e-6
            p2 = jnp.abs(plsc.load_gather(lv, [splat(2), hst, hss, rv])) + 1e-6
            quad = p0 * t1 * t1 + p1 * t2 * t2 + p2 * t3 * t3
            accp = accp + quad * kms
            pprod = pprod * jnp.where(kms > 0, p0 * p1 * p2, 1.0)

            xq = None
            ldet = None
            for c in range(3):
                vc = plsc.load_gather(vv, [splat(c), hst, hss, rv]) + 1e-8
                dxc = plsc.load_gather(xv, [splat(c), hst, hss, rv]) - gt[c]
                term = dxc * dxc / vc
                xq = term if xq is None else xq + term
                ldet = vc if ldet is None else ldet * vc
            accg = accg + xq * kms
            dprod = dprod * jnp.where(kms > 0, ldet, 1.0)

            accw = accw + plsc.load_gather(wv, [hst, hss, rv]) * kms
        accp = accp - 0.5 * _ln(pprod)
        accg = accg + _ln(dprod)
        return (accp, accg, accw)

    zero = jnp.zeros((16,), jnp.float32)

    def chunk_body(ct, accs):
        accp, accg, accb, accw = accs
        bt = wid * TPW + ct
        grp_a = [pltpu.make_async_copy(q_h.at[:, bt, :, :], qv, semA),
                 pltpu.make_async_copy(x_h.at[:, :, bt, :, :], xv, semA),
                 pltpu.make_async_copy(gq_h.at[bt, :, :], gqv, semA),
                 pltpu.make_async_copy(gt_h.at[bt, :, :], gtv, semA)]
        grp_b = [pltpu.make_async_copy(l_h.at[:, :, bt, :, :], lv, semB),
                 pltpu.make_async_copy(v_h.at[:, :, bt, :, :], vv, semB),
                 pltpu.make_async_copy(w_h.at[:, bt, :, :], wv, semB)]
        for cp in grp_a + grp_b:
            cp.start()
        for cp in grp_a:
            cp.wait()
        lax.fori_loop(0, 8, phase_a, 0)
        lax.fori_loop(0, R, phase_b, 0, unroll=4)
        for cp in grp_b:
            cp.wait()
        accp, accg, accw = lax.fori_loop(0, 8, phase_c, (accp, accg, accw))
        accb = lax.fori_loop(0, 8, phase_w, accb)
        return (accp, accg, accb, accw)

    accp, accg, accb, accw = lax.fori_loop(
        0, TPW, chunk_body, (zero, zero, zero, zero))

    outv[pl.ds(0, 16)] = accp
    outv[pl.ds(16, 16)] = accg
    outv[pl.ds(32, 16)] = accb
    outv[pl.ds(48, 16)] = accw
    pltpu.sync_copy(outv, out_h.at[pl.ds(wid * 64, 64)])


def _make_call():
    mesh = plsc.VectorSubcoreMesh(core_axis_name="c", subcore_axis_name="s",
                                  num_cores=NC, num_subcores=NS)
    scratch = [
        pltpu.VMEM((64, 4, 128), jnp.float32),
        pltpu.VMEM((3, 8, 8, 128), jnp.float32),
        pltpu.VMEM((3, 8, 8, 128), jnp.float32),
        pltpu.VMEM((3, 8, 8, 128), jnp.float32),
        pltpu.VMEM((8, 8, 128), jnp.float32),
        pltpu.VMEM((4, 128), jnp.float32),
        pltpu.VMEM((4, 128), jnp.float32),
        pltpu.VMEM((16,), jnp.float32),
        pltpu.VMEM((64,), jnp.float32),
        pltpu.VMEM((H * PITCH,), jnp.float32),
        pltpu.VMEM((8 * PITCH,), jnp.int32),
        pltpu.SemaphoreType.DMA,
        pltpu.SemaphoreType.DMA,
    ]
    return pl.kernel(
        _body,
        out_type=jax.ShapeDtypeStruct((NW * 64,), jnp.float32),
        mesh=mesh,
        scratch_types=scratch,
        compiler_params=pltpu.CompilerParams(needs_layout_passes=False),
    )


_sc_call = _make_call()


def kernel(pred_q, pred_l, weights, gt_q, pred_x, pred_var, gt_t, k):
    km = (jnp.arange(16) < jnp.minimum(jnp.asarray(k, jnp.int32), 8)).astype(jnp.float32)
    qP = pred_q.reshape(128, 128, 64, 4).transpose(2, 0, 3, 1)
    lP = pred_l.reshape(128, 128, 8, 8, 3).transpose(4, 2, 0, 3, 1)
    xP = pred_x.reshape(128, 128, 8, 8, 3).transpose(4, 2, 0, 3, 1)
    vP = pred_var.reshape(128, 128, 8, 8, 3).transpose(4, 2, 0, 3, 1)
    wP = weights.reshape(128, 128, 8, 8).transpose(2, 0, 3, 1)
    gqP = gt_q.reshape(128, 128, 4).transpose(0, 2, 1)
    gtP = jnp.pad(gt_t, ((0, 0), (0, 1))).reshape(128, 128, 4).transpose(0, 2, 1)
    out = _sc_call(qP, lP, wP, gqP, xP, vP, gtP, km)
    s = jnp.sum(out.reshape(NW, 4, 16), axis=(0, 2))
    nk = jnp.minimum(jnp.asarray(k, jnp.float32), 8.0)
    loss = s[0] / B + C_BING * nk
    gloss = 0.5 * (s[1] / B + LOG2PI3 * nk)
    weight_loss = (s[2] - s[3]) / (B * H)
    return (loss, weight_loss, gloss)


# speedup vs baseline: 1.1162x; 1.1162x over previous
import jax
import jax.numpy as jnp
from jax import lax
from jax.experimental import pallas as pl
from jax.experimental.pallas import tpu as pltpu
from jax.experimental.pallas import tpu_sc as plsc

B = 16384
H = 64
NC, NS = 2, 16
NW = NC * NS
TPW = (B // 128) // NW
R = 128
PITCH = 129

LN2 = 0.6931471805599453
SQRT2 = 1.4142135623730951
C_BING = 0.6931471805599453 + 1.5 * 1.1447298858494002
LOG2PI3 = 3.0 * 1.8378770664093453


def _ln(x):
    bits = plsc.bitcast(x, jnp.int32)
    e = (bits >> 23) - 127
    m = plsc.bitcast((bits & 0x007FFFFF) | 0x3F800000, jnp.float32)
    big = m > SQRT2
    m = jnp.where(big, m * 0.5, m)
    ef = e.astype(jnp.float32) + jnp.where(big, 1.0, 0.0)
    t = (m - 1.0) / (m + 1.0)
    t2 = t * t
    poly = t * (2.0 + t2 * (0.66666667 + t2 * (0.4 + t2 * (0.28571429 + t2 * 0.22222222))))
    return ef * LN2 + poly


def _rsqrt(x):
    bits = plsc.bitcast(x, jnp.int32)
    y = plsc.bitcast(0x5F3759DF - (bits >> 1), jnp.float32)
    for _ in range(3):
        y = y * (1.5 - 0.5 * x * y * y)
    return y


def _body(q_h, l_h, w_h, gq_h, x_h, v_h, gt_h, km_h, out_h, *scr):
    qv, lv, xv, vv, wv, gqv, gtv, kmv, outv, lbuf, selv, semA, semB = scr

    cid = lax.axis_index("c")
    sid = lax.axis_index("s")
    wid = sid * NC + cid

    pltpu.sync_copy(km_h, kmv)
    km = kmv[...]

    iota = lax.broadcasted_iota(jnp.int32, (16,), 0)
    iotap = iota * PITCH
    lane_lt8 = iota < 8
    zero16 = jnp.zeros((16,), jnp.int32)

    def splat(i):
        return zero16 + i

    def phase_a(m, carry):
        rv = m * 16 + iota
        gq = [plsc.load_gather(gqv, [splat(c), rv]) for c in range(4)]
        gt = [plsc.load_gather(gtv, [splat(c), rv]) for c in range(3)]
        for h in range(H):
            ht, hs = h >> 3, h & 7
            dq = [jnp.abs(plsc.load_gather(qv, [splat(h), splat(c), rv]) - gq[c])
                  for c in range(4)]
            dx = [jnp.abs(plsc.load_gather(xv, [splat(c), splat(ht), splat(hs), rv]) - gt[c])
                  for c in range(3)]
            acc = ((dq[0] + dq[1]) + (dq[2] + dq[3])) + ((dx[0] + dx[1]) + dx[2])
            plsc.store_scatter(lbuf, [rv + h * PITCH], acc)
        return carry

    def phase_w(m, accb):
        rv = m * 16 + iota
        uprod = jnp.zeros((16,), jnp.float32) + 1.0
        relu = jnp.zeros((16,), jnp.float32)
        for h in range(H):
            ht, hs = h >> 3, h & 7
            wj = plsc.load_gather(wv, [splat(ht), splat(hs), rv])
            relu = relu + 0.5 * (wj + jnp.abs(wj))
            uprod = uprod * (1.0 + jnp.exp(-jnp.abs(wj)))
            if (h & 15) == 15:
                accb = accb + _ln(uprod)
                uprod = jnp.zeros((16,), jnp.float32) + 1.0
        return accb + relu

    def phase_b(r, carry):
        tk = None
        tv = None
        for j in range(4):
            keys = plsc.load_gather(lbuf, [iotap + (j * 16 * PITCH + r)])
            sk, sv = plsc.sort_key_val(keys, iota + 16 * j)
            if tk is None:
                tk, tv = sk, sv
            else:
                ck = jnp.where(lane_lt8, tk, jnp.flip(sk))
                cv = jnp.where(lane_lt8, tv, jnp.flip(sv))
                tk, tv = plsc.sort_key_val(ck, cv)
        plsc.store_scatter(selv, [iotap + r], tv, mask=lane_lt8)
        return carry

    def phase_c(m, accs):
        accp, accg, accw = accs
        rv = m * 16 + iota
        gq = [plsc.load_gather(gqv, [splat(c), rv]) for c in range(4)]
        gt = [plsc.load_gather(gtv, [splat(c), rv]) for c in range(3)]
        pprod = jnp.zeros((16,), jnp.float32) + 1.0
        dprod = jnp.zeros((16,), jnp.float32) + 1.0
        for s in range(8):
            kms = km[s]
            hsel = plsc.load_gather(selv, [rv + s * PITCH])
            hst = hsel >> 3
            hss = hsel & 7

            qs = [plsc.load_gather(qv, [hsel, splat(c), rv]) for c in range(4)]
            sq = qs[0] * qs[0] + qs[1] * qs[1] + qs[2] * qs[2] + qs[3] * qs[3]
            rn = _rsqrt(sq)
            a_ = qs[0] * rn
            b_ = qs[1] * rn
            c_ = qs[2] * rn
            d_ = qs[3] * rn
            t1 = gq[1] * a_ - gq[0] * b_ + gq[3] * c_ - gq[2] * d_
            t2 = gq[2] * a_ - gq[3] * b_ - gq[0] * c_ + gq[1] * d_
            t3 = gq[3] * a_ + gq[2] * b_ - gq[1] * c_ - gq[0] * d_
            p0 = jnp.abs(plsc.load_gather(lv, [splat(0), hst, hss, rv])) + 1e-6
            p1 = jnp.abs(plsc.load_gather(lv, [splat(1), hst, hss, rv])) + 1e-6
            p2 = jnp.abs(plsc.load_gather(lv, [splat(2), hst, hss, rv])) + 1e-6
            quad = p0 * t1 * t1 + p1 * t2 * t2 + p2 * t3 * t3
            accp = accp + quad * kms
            pprod = pprod * jnp.where(kms > 0, p0 * p1 * p2, 1.0)

            xq = None
            ldet = None
            for c in range(3):
                vc = plsc.load_gather(vv, [splat(c), hst, hss, rv]) + 1e-8
                dxc = plsc.load_gather(xv, [splat(c), hst, hss, rv]) - gt[c]
                term = dxc * dxc / vc
                xq = term if xq is None else xq + term
                ldet = vc if ldet is None else ldet * vc
            accg = accg + xq * kms
            dprod = dprod * jnp.where(kms > 0, ldet, 1.0)

            accw = accw + plsc.load_gather(wv, [hst, hss, rv]) * kms
        accp = accp - 0.5 * _ln(pprod)
        accg = accg + _ln(dprod)
        return (accp, accg, accw)

    zero = jnp.zeros((16,), jnp.float32)

    def grp_a_copies(bt):
        return [pltpu.make_async_copy(q_h.at[:, bt, :, :], qv, semA),
                pltpu.make_async_copy(x_h.at[:, :, bt, :, :], xv, semA),
                pltpu.make_async_copy(gq_h.at[bt, :, :], gqv, semA),
                pltpu.make_async_copy(gt_h.at[bt, :, :], gtv, semA)]

    def chunk_body(ct, accs):
        accp, accg, accb, accw = accs
        bt = wid * TPW + ct
        grp_b = [pltpu.make_async_copy(l_h.at[:, :, bt, :, :], lv, semB),
                 pltpu.make_async_copy(v_h.at[:, :, bt, :, :], vv, semB),
                 pltpu.make_async_copy(w_h.at[:, bt, :, :], wv, semB)]
        for cp in grp_b:
            cp.start()
        for cp in grp_a_copies(bt):
            cp.wait()
        lax.fori_loop(0, 8, phase_a, 0)
        lax.fori_loop(0, R, phase_b, 0, unroll=2)
        for cp in grp_b:
            cp.wait()
        accp, accg, accw = lax.fori_loop(0, 8, phase_c, (accp, accg, accw))

        @pl.when(ct + 1 < TPW)
        def _():
            for cp in grp_a_copies(bt + 1):
                cp.start()

        accb = lax.fori_loop(0, 8, phase_w, accb)
        return (accp, accg, accb, accw)

    for cp in grp_a_copies(wid * TPW):
        cp.start()
    accp, accg, accb, accw = lax.fori_loop(
        0, TPW, chunk_body, (zero, zero, zero, zero))

    outv[pl.ds(0, 16)] = accp
    outv[pl.ds(16, 16)] = accg
    outv[pl.ds(32, 16)] = accb
    outv[pl.ds(48, 16)] = accw
    pltpu.sync_copy(outv, out_h.at[pl.ds(wid * 64, 64)])


def _make_call():
    mesh = plsc.VectorSubcoreMesh(core_axis_name="c", subcore_axis_name="s",
                                  num_cores=NC, num_subcores=NS)
    scratch = [
        pltpu.VMEM((64, 4, 128), jnp.float32),
        pltpu.VMEM((3, 8, 8, 128), jnp.float32),
        pltpu.VMEM((3, 8, 8, 128), jnp.float32),
        pltpu.VMEM((3, 8, 8, 128), jnp.float32),
        pltpu.VMEM((8, 8, 128), jnp.float32),
        pltpu.VMEM((4, 128), jnp.float32),
        pltpu.VMEM((4, 128), jnp.float32),
        pltpu.VMEM((16,), jnp.float32),
        pltpu.VMEM((64,), jnp.float32),
        pltpu.VMEM((H * PITCH,), jnp.float32),
        pltpu.VMEM((8 * PITCH,), jnp.int32),
        pltpu.SemaphoreType.DMA,
        pltpu.SemaphoreType.DMA,
    ]
    return pl.kernel(
        _body,
        out_type=jax.ShapeDtypeStruct((NW * 64,), jnp.float32),
        mesh=mesh,
        scratch_types=scratch,
        compiler_params=pltpu.CompilerParams(needs_layout_passes=False),
    )


_sc_call = _make_call()


def kernel(pred_q, pred_l, weights, gt_q, pred_x, pred_var, gt_t, k):
    km = (jnp.arange(16) < jnp.minimum(jnp.asarray(k, jnp.int32), 8)).astype(jnp.float32)
    qP = pred_q.reshape(128, 128, 64, 4).transpose(2, 0, 3, 1)
    lP = pred_l.reshape(128, 128, 8, 8, 3).transpose(4, 2, 0, 3, 1)
    xP = pred_x.reshape(128, 128, 8, 8, 3).transpose(4, 2, 0, 3, 1)
    vP = pred_var.reshape(128, 128, 8, 8, 3).transpose(4, 2, 0, 3, 1)
    wP = weights.reshape(128, 128, 8, 8).transpose(2, 0, 3, 1)
    gqP = gt_q.reshape(128, 128, 4).transpose(0, 2, 1)
    gtP = jnp.pad(gt_t, ((0, 0), (0, 1))).reshape(128, 128, 4).transpose(0, 2, 1)
    out = _sc_call(qP, lP, wP, gqP, xP, vP, gtP, km)
    s = jnp.sum(out.reshape(NW, 4, 16), axis=(0, 2))
    nk = jnp.minimum(jnp.asarray(k, jnp.float32), 8.0)
    loss = s[0] / B + C_BING * nk
    gloss = 0.5 * (s[1] / B + LOG2PI3 * nk)
    weight_loss = (s[2] - s[3]) / (B * H)
    return (loss, weight_loss, gloss)
